# baseline (speedup 1.0000x reference)
import functools

import jax
import jax.numpy as jnp
from jax import lax
from jax.experimental import pallas as pl
from jax.experimental.pallas import tpu as pltpu
from jax.experimental.pallas import tpu_sc as plsc

B, H, S, D = 8, 16, 4096, 128
Q = 32
P0 = 1024
E = P0 + Q
BH = B * H

NC, NS = 2, 16
NW = NC * NS
PPW = BH // NW
CHUNK = 256
NCHUNK = P0 // CHUNK

_mesh = plsc.VectorSubcoreMesh(
    core_axis_name="c", subcore_axis_name="s", num_cores=NC, num_subcores=NS)


NBUF = 3


def _sc_body(kc, kn, vc, vn, ok, ov, buf0, buf1, shared, l0, l1, l2, s0, s1, s2):
    c = lax.axis_index("c")
    s = lax.axis_index("s")
    base = (s * NC + c) * PPW

    bufs = (buf0, buf1, shared.at[s, 0])
    lsems = (l0, l1, l2)
    ssems = (s0, s1, s2)

    jobs = []
    for p_local in range(PPW):
        for which in range(2):
            for ci in range(NCHUNK):
                jobs.append((p_local, which, ci))
            jobs.append((p_local, which, None))

    tensors = ((kc, kn, ok), (vc, vn, ov))

    def mk(g):
        p_local, which, ci = jobs[g]
        tin, tnew, tout = tensors[which]
        p = base + p_local
        b = g % NBUF
        if ci is None:
            src = tnew.at[p]
            dst = tout.at[p, pl.ds(P0, Q), :]
            rows = Q
        else:
            src = tin.at[p, pl.ds(ci * CHUNK, CHUNK), :]
            dst = tout.at[p, pl.ds(ci * CHUNK, CHUNK), :]
            rows = CHUNK
        ld = pltpu.make_async_copy(src, bufs[b].at[pl.ds(0, rows)], lsems[b])
        st = pltpu.make_async_copy(bufs[b].at[pl.ds(0, rows)], dst, ssems[b])
        return ld, st

    n = len(jobs)
    prev_store = [None] * NBUF
    pending = None
    for g in range(n):
        b = g % NBUF
        ld, st = mk(g)
        if prev_store[b] is not None:
            prev_store[b].wait()
        ld.start()
        if pending is not None:
            pld, pst = pending
            pld.wait()
            pst.start()
            prev_store[(g - 1) % NBUF] = pst
        pending = (ld, st)
    pld, pst = pending
    pld.wait()
    pst.start()
    prev_store[(n - 1) % NBUF] = pst
    for d in prev_store:
        if d is not None:
            d.wait()


@functools.partial(
    pl.kernel,
    out_type=[jax.ShapeDtypeStruct((BH, E, D), jnp.float32)] * 2,
    mesh=_mesh,
    scratch_types=(
        [pltpu.VMEM((CHUNK, D), jnp.float32)] * 2
        + [pltpu.VMEM_SHARED((NS, 2, CHUNK, D), jnp.float32)]
        + [pltpu.SemaphoreType.DMA] * (2 * NBUF)
    ),
)
def _sc_copy(kc, kn, vc, vn, ok, ov, *scratch):
    _sc_body(kc, kn, vc, vn, ok, ov, *scratch)


def kernel(k_new, v_new, k_cache, v_cache, start_pos):
    del start_pos
    kc = k_cache.reshape(BH, S, D)
    vc = v_cache.reshape(BH, S, D)
    kn = k_new.reshape(BH, Q, D)
    vn = v_new.reshape(BH, Q, D)
    ok, ov = _sc_copy(kc, kn, vc, vn)
    return ok.reshape(B, H, E, D), ov.reshape(B, H, E, D)


# speedup vs baseline: 1.0475x; 1.0475x over previous
import functools

import jax
import jax.numpy as jnp
from jax import lax
from jax.experimental import pallas as pl
from jax.experimental.pallas import tpu as pltpu
from jax.experimental.pallas import tpu_sc as plsc

B, H, S, D = 8, 16, 4096, 128
Q = 32
P0 = 1024
E = P0 + Q
BH = B * H

NC, NS = 2, 16
NW = NC * NS
PPW = BH // NW
CHUNK = 256
NCHUNK = P0 // CHUNK

_mesh = plsc.VectorSubcoreMesh(
    core_axis_name="c", subcore_axis_name="s", num_cores=NC, num_subcores=NS)


NBUF = 3


def _sc_body(kc, kn, vc, vn, ok, ov, buf0, buf1, shared, l0, l1, l2, s0, s1, s2):
    c = lax.axis_index("c")
    s = lax.axis_index("s")
    base = (s * NC + c) * PPW

    bufs = (buf0, shared.at[s, 0], shared.at[s, 1])
    lsems = (l0, l1, l2)
    ssems = (s0, s1, s2)

    jobs = []
    for p_local in range(PPW):
        for which in range(2):
            for ci in range(NCHUNK):
                jobs.append((p_local, which, ci))
            jobs.append((p_local, which, None))

    tensors = ((kc, kn, ok), (vc, vn, ov))

    def mk(g):
        p_local, which, ci = jobs[g]
        tin, tnew, tout = tensors[which]
        p = base + p_local
        b = g % NBUF
        if ci is None:
            src = tnew.at[p]
            dst = tout.at[p, pl.ds(P0, Q), :]
            rows = Q
        else:
            src = tin.at[p, pl.ds(ci * CHUNK, CHUNK), :]
            dst = tout.at[p, pl.ds(ci * CHUNK, CHUNK), :]
            rows = CHUNK
        ld = pltpu.make_async_copy(src, bufs[b].at[pl.ds(0, rows)], lsems[b])
        st = pltpu.make_async_copy(bufs[b].at[pl.ds(0, rows)], dst, ssems[b])
        return ld, st

    n = len(jobs)
    prev_store = [None] * NBUF
    pending = None
    for g in range(n):
        b = g % NBUF
        ld, st = mk(g)
        if prev_store[b] is not None:
            prev_store[b].wait()
        ld.start()
        if pending is not None:
            pld, pst = pending
            pld.wait()
            pst.start()
            prev_store[(g - 1) % NBUF] = pst
        pending = (ld, st)
    pld, pst = pending
    pld.wait()
    pst.start()
    prev_store[(n - 1) % NBUF] = pst
    for d in prev_store:
        if d is not None:
            d.wait()


@functools.partial(
    pl.kernel,
    out_type=[jax.ShapeDtypeStruct((BH, E, D), jnp.float32)] * 2,
    mesh=_mesh,
    scratch_types=(
        [pltpu.VMEM((CHUNK, D), jnp.float32)] * 2
        + [pltpu.VMEM_SHARED((NS, 2, CHUNK, D), jnp.float32)]
        + [pltpu.SemaphoreType.DMA] * (2 * NBUF)
    ),
)
def _sc_copy(kc, kn, vc, vn, ok, ov, *scratch):
    _sc_body(kc, kn, vc, vn, ok, ov, *scratch)


def kernel(k_new, v_new, k_cache, v_cache, start_pos):
    del start_pos
    kc = k_cache.reshape(BH, S, D)
    vc = v_cache.reshape(BH, S, D)
    kn = k_new.reshape(BH, Q, D)
    vn = v_new.reshape(BH, Q, D)
    ok, ov = _sc_copy(kc, kn, vc, vn)
    return ok.reshape(B, H, E, D), ov.reshape(B, H, E, D)


# speedup vs baseline: 1.0489x; 1.0013x over previous
import functools

import jax
import jax.numpy as jnp
from jax import lax
from jax.experimental import pallas as pl
from jax.experimental.pallas import tpu as pltpu
from jax.experimental.pallas import tpu_sc as plsc

B, H, S, D = 8, 16, 4096, 128
Q = 32
P0 = 1024
E = P0 + Q
BH = B * H

NC, NS = 2, 16
NW = NC * NS
PPW = BH // NW
CHUNK = 256
NCHUNK = P0 // CHUNK

_mesh = plsc.VectorSubcoreMesh(
    core_axis_name="c", subcore_axis_name="s", num_cores=NC, num_subcores=NS)


NBUF = 4


def _sc_body(kc, kn, vc, vn, ok, ov, buf0, buf1, shared, l0, l1, l2, l3, s0, s1, s2, s3):
    c = lax.axis_index("c")
    s = lax.axis_index("s")
    base = (s * NC + c) * PPW

    bufs = (buf0, shared.at[s, 0], shared.at[s, 1], shared.at[s, 2])
    lsems = (l0, l1, l2, l3)
    ssems = (s0, s1, s2, s3)

    jobs = []
    for p_local in range(PPW):
        for which in range(2):
            for ci in range(NCHUNK):
                jobs.append((p_local, which, ci))
            jobs.append((p_local, which, None))

    tensors = ((kc, kn, ok), (vc, vn, ov))

    def mk(g):
        p_local, which, ci = jobs[g]
        tin, tnew, tout = tensors[which]
        p = base + p_local
        b = g % NBUF
        if ci is None:
            src = tnew.at[p]
            dst = tout.at[p, pl.ds(P0, Q), :]
            rows = Q
        else:
            src = tin.at[p, pl.ds(ci * CHUNK, CHUNK), :]
            dst = tout.at[p, pl.ds(ci * CHUNK, CHUNK), :]
            rows = CHUNK
        ld = pltpu.make_async_copy(src, bufs[b].at[pl.ds(0, rows)], lsems[b])
        st = pltpu.make_async_copy(bufs[b].at[pl.ds(0, rows)], dst, ssems[b])
        return ld, st

    n = len(jobs)
    prev_store = [None] * NBUF
    pending = None
    for g in range(n):
        b = g % NBUF
        ld, st = mk(g)
        if prev_store[b] is not None:
            prev_store[b].wait()
        ld.start()
        if pending is not None:
            pld, pst = pending
            pld.wait()
            pst.start()
            prev_store[(g - 1) % NBUF] = pst
        pending = (ld, st)
    pld, pst = pending
    pld.wait()
    pst.start()
    prev_store[(n - 1) % NBUF] = pst
    for d in prev_store:
        if d is not None:
            d.wait()


@functools.partial(
    pl.kernel,
    out_type=[jax.ShapeDtypeStruct((BH, E, D), jnp.float32)] * 2,
    mesh=_mesh,
    scratch_types=(
        [pltpu.VMEM((CHUNK, D), jnp.float32)] * 2
        + [pltpu.VMEM_SHARED((NS, 3, CHUNK, D), jnp.float32)]
        + [pltpu.SemaphoreType.DMA] * (2 * NBUF)
    ),
)
def _sc_copy(kc, kn, vc, vn, ok, ov, *scratch):
    _sc_body(kc, kn, vc, vn, ok, ov, *scratch)


def kernel(k_new, v_new, k_cache, v_cache, start_pos):
    del start_pos
    kc = k_cache.reshape(BH, S, D)
    vc = v_cache.reshape(BH, S, D)
    kn = k_new.reshape(BH, Q, D)
    vn = v_new.reshape(BH, Q, D)
    ok, ov = _sc_copy(kc, kn, vc, vn)
    return ok.reshape(B, H, E, D), ov.reshape(B, H, E, D)


# speedup vs baseline: 1.0526x; 1.0035x over previous
import functools

import jax
import jax.numpy as jnp
from jax import lax
from jax.experimental import pallas as pl
from jax.experimental.pallas import tpu as pltpu
from jax.experimental.pallas import tpu_sc as plsc

B, H, S, D = 8, 16, 4096, 128
Q = 32
P0 = 1024
E = P0 + Q
BH = B * H

NC, NS = 2, 16
NW = NC * NS
PPW = BH // NW
CHUNK = 256
NCHUNK = P0 // CHUNK

_mesh = plsc.VectorSubcoreMesh(
    core_axis_name="c", subcore_axis_name="s", num_cores=NC, num_subcores=NS)


NBUF = 3


def _sc_body(kc, kn, vc, vn, ok, ov, buf0, buf1, shared, l0, l1, l2, s0, s1, s2):
    c = lax.axis_index("c")
    s = lax.axis_index("s")
    base = (s * NC + c) * PPW

    bufs = (shared.at[s, 0], shared.at[s, 1], shared.at[s, 2])
    lsems = (l0, l1, l2)
    ssems = (s0, s1, s2)

    jobs = []
    for p_local in range(PPW):
        for which in range(2):
            for ci in range(NCHUNK):
                jobs.append((p_local, which, ci))
            jobs.append((p_local, which, None))

    tensors = ((kc, kn, ok), (vc, vn, ov))

    def mk(g):
        p_local, which, ci = jobs[g]
        tin, tnew, tout = tensors[which]
        p = base + p_local
        b = g % NBUF
        if ci is None:
            src = tnew.at[p]
            dst = tout.at[p, pl.ds(P0, Q), :]
            rows = Q
        else:
            src = tin.at[p, pl.ds(ci * CHUNK, CHUNK), :]
            dst = tout.at[p, pl.ds(ci * CHUNK, CHUNK), :]
            rows = CHUNK
        ld = pltpu.make_async_copy(src, bufs[b].at[pl.ds(0, rows)], lsems[b])
        st = pltpu.make_async_copy(bufs[b].at[pl.ds(0, rows)], dst, ssems[b])
        return ld, st

    n = len(jobs)
    prev_store = [None] * NBUF
    pending = None
    for g in range(n):
        b = g % NBUF
        ld, st = mk(g)
        if prev_store[b] is not None:
            prev_store[b].wait()
        ld.start()
        if pending is not None:
            pld, pst = pending
            pld.wait()
            pst.start()
            prev_store[(g - 1) % NBUF] = pst
        pending = (ld, st)
    pld, pst = pending
    pld.wait()
    pst.start()
    prev_store[(n - 1) % NBUF] = pst
    for d in prev_store:
        if d is not None:
            d.wait()


@functools.partial(
    pl.kernel,
    out_type=[jax.ShapeDtypeStruct((BH, E, D), jnp.float32)] * 2,
    mesh=_mesh,
    scratch_types=(
        [pltpu.VMEM((CHUNK, D), jnp.float32)] * 2
        + [pltpu.VMEM_SHARED((NS, 3, CHUNK, D), jnp.float32)]
        + [pltpu.SemaphoreType.DMA] * (2 * NBUF)
    ),
)
def _sc_copy(kc, kn, vc, vn, ok, ov, *scratch):
    _sc_body(kc, kn, vc, vn, ok, ov, *scratch)


def kernel(k_new, v_new, k_cache, v_cache, start_pos):
    del start_pos
    kc = k_cache.reshape(BH, S, D)
    vc = v_cache.reshape(BH, S, D)
    kn = k_new.reshape(BH, Q, D)
    vn = v_new.reshape(BH, Q, D)
    ok, ov = _sc_copy(kc, kn, vc, vn)
    return ok.reshape(B, H, E, D), ov.reshape(B, H, E, D)
